# baseline (speedup 1.0000x reference)
import functools

import jax
import jax.numpy as jnp
from jax import lax
from jax.experimental import pallas as pl
from jax.experimental.pallas import tpu as pltpu
from jax.experimental.pallas import tpu_sc as plsc

NC = 2
NS = 16
L = 16
NW = NC * NS

B = 16384
K = 10
D = 64
VOCAB_PAIRS = 500000

BPW = B // NW
CHUNK = 32
NCHUNK = BPW // CHUNK
NGRP = CHUNK // L
NSROWS = CHUNK * K
NSU = 4
NSUR = NSROWS // NSU


def _sc_body(cen_q, cen_p, ctx_q, ctx_p, ns_q, ns_p, wc_hbm, wx_hbm, out_hbm,
             qcen_v, pcen_v, qctx_v, pctx_v, qns_v, pns_v, score_v,
             c_rows0, c_rows1, x_rows0, x_rows1, n_rows0, n_rows1,
             sem0, sem1):
  wid = lax.axis_index("s") * NC + lax.axis_index("c")

  pltpu.sync_copy(cen_q.at[wid], qcen_v)
  pltpu.sync_copy(cen_p.at[wid], pcen_v)
  pltpu.sync_copy(ctx_q.at[wid], qctx_v)
  pltpu.sync_copy(ctx_p.at[wid], pctx_v)
  pltpu.sync_copy(ns_q.at[wid], qns_v)
  pltpu.sync_copy(ns_p.at[wid], pns_v)

  bufs = ((c_rows0, x_rows0, n_rows0, sem0),
          (c_rows1, x_rows1, n_rows1, sem1))

  def issue(g):
    c_b, x_b, n_b, sem = bufs[g % 2]
    cps = [
        pltpu.async_copy(wc_hbm.at[qcen_v.at[pl.ds(g * CHUNK, CHUNK)]],
                         c_b, sem),
        pltpu.async_copy(wx_hbm.at[qctx_v.at[pl.ds(g * CHUNK, CHUNK)]],
                         x_b, sem),
    ]
    for u in range(NSU):
      cps.append(pltpu.async_copy(
          wx_hbm.at[qns_v.at[pl.ds(g * NSROWS + u * NSUR, NSUR)]],
          n_b.at[pl.ds(u * NSUR, NSUR)], sem))
    return cps

  iota = lax.iota(jnp.int32, L)
  pending = issue(0)

  for g in range(NCHUNK):
    nxt = issue(g + 1) if g + 1 < NCHUNK else None
    for cp in pending:
      cp.wait()
    pending = nxt

    c_b, x_b, n_b, _ = bufs[g % 2]
    for grp in range(NGRP):
      base = g * CHUNK + grp * L
      row = grp * L + iota
      pcen = pcen_v[pl.ds(base, L)]
      pctx = pctx_v[pl.ds(base, L)]
      nrow = [row * K + k for k in range(K)]
      npar = [plsc.load_gather(pns_v, [(base + iota) * K + k])
              for k in range(K)]

      def body(d, accs):
        cv = plsc.load_gather(c_b, [row, pcen + d])
        xv = plsc.load_gather(x_b, [row, pctx + d])
        new = [accs[0] + cv * xv]
        for k in range(K):
          nv = plsc.load_gather(n_b, [nrow[k], npar[k] + d])
          new.append(accs[k + 1] + cv * nv)
        return tuple(new)

      accs = lax.fori_loop(
          0, D, body, tuple(jnp.zeros((L,), jnp.float32) for _ in range(K + 1)))

      score_v[0, pl.ds(base, L)] = -accs[0]
      for k in range(K):
        score_v[1 + k, pl.ds(base, L)] = accs[k + 1]

  pltpu.sync_copy(score_v, out_hbm.at[wid])


def _tc_body(s_ref, o_ref):
  x = s_ref[...]
  o_ref[0, 0] = jnp.sum(jnp.maximum(x, 0.0) +
                        jnp.log1p(jnp.exp(-jnp.abs(x))))


@jax.jit
def kernel(center, context, ns, W_center, W_context):
  cen = center.astype(jnp.int32)
  ctx = context.astype(jnp.int32)
  nsf = ns.astype(jnp.int32).reshape(-1)

  cen_q = (cen >> 1).reshape(NW, BPW)
  cen_p = ((cen & 1) << 6).reshape(NW, BPW)
  ctx_q = (ctx >> 1).reshape(NW, BPW)
  ctx_p = ((ctx & 1) << 6).reshape(NW, BPW)
  ns_q = (nsf >> 1).reshape(NW, BPW * K)
  ns_p = ((nsf & 1) << 6).reshape(NW, BPW * K)

  wc2 = W_center.reshape(VOCAB_PAIRS, 2 * D)
  wx2 = W_context.reshape(VOCAB_PAIRS, 2 * D)

  mesh = plsc.VectorSubcoreMesh(core_axis_name="c", subcore_axis_name="s")
  scores = pl.kernel(
      _sc_body,
      out_type=jax.ShapeDtypeStruct((NW, 1 + K, BPW), jnp.float32),
      mesh=mesh,
      compiler_params=pltpu.CompilerParams(needs_layout_passes=False),
      scratch_types=[
          pltpu.VMEM((BPW,), jnp.int32),
          pltpu.VMEM((BPW,), jnp.int32),
          pltpu.VMEM((BPW,), jnp.int32),
          pltpu.VMEM((BPW,), jnp.int32),
          pltpu.VMEM((BPW * K,), jnp.int32),
          pltpu.VMEM((BPW * K,), jnp.int32),
          pltpu.VMEM((1 + K, BPW), jnp.float32),
          pltpu.VMEM((CHUNK, 2 * D), jnp.float32),
          pltpu.VMEM((CHUNK, 2 * D), jnp.float32),
          pltpu.VMEM((CHUNK, 2 * D), jnp.float32),
          pltpu.VMEM((CHUNK, 2 * D), jnp.float32),
          pltpu.VMEM((NSROWS, 2 * D), jnp.float32),
          pltpu.VMEM((NSROWS, 2 * D), jnp.float32),
          pltpu.SemaphoreType.DMA,
          pltpu.SemaphoreType.DMA,
      ],
  )(cen_q, cen_p, ctx_q, ctx_p, ns_q, ns_p, wc2, wx2)

  loss = pl.pallas_call(
      _tc_body,
      out_shape=jax.ShapeDtypeStruct((1, 1), jnp.float32),
      out_specs=pl.BlockSpec(memory_space=pltpu.SMEM),
  )(scores.reshape(NW * (1 + K), BPW))
  return loss[0, 0]
